# Initial kernel scaffold; baseline (speedup 1.0000x reference)
import jax
import jax.numpy as jnp
from jax.experimental import pallas as pl


def kernel(positions, table):
    raise NotImplementedError("write your pallas kernel here")


# speedup vs baseline: 2.3756x; 2.3756x over previous
import functools

import jax
import jax.numpy as jnp
from jax import lax
from jax.experimental import pallas as pl
from jax.experimental.pallas import tpu as pltpu
from jax.experimental.pallas import tpu_sc as plsc

_MAX_LEN = 8192
_CHUNK = 32


@functools.lru_cache(maxsize=None)
def _make_gather(B, V, D):
    info = plsc.get_sparse_core_info()
    nc, ns = info.num_cores, info.num_subcores
    nw = nc * ns
    b_per_w = B // nw
    n_chunks = b_per_w // _CHUNK
    assert b_per_w * nw == B and n_chunks * _CHUNK == b_per_w and n_chunks % 2 == 0

    mesh = plsc.VectorSubcoreMesh(core_axis_name="c", subcore_axis_name="s")

    @functools.partial(
        pl.kernel,
        mesh=mesh,
        out_type=jax.ShapeDtypeStruct((B, D), jnp.float32),
        scratch_types=[
            pltpu.VMEM((b_per_w,), jnp.int32),
            pltpu.VMEM((2, _CHUNK, D), jnp.float32),
            pltpu.SemaphoreType.DMA,
            pltpu.SemaphoreType.DMA,
        ],
    )
    def gather_kernel(idx_hbm, table_hbm, out_hbm, idx_v, rows_v, sem0, sem1):
        sems = (sem0, sem1)
        wid = lax.axis_index("s") * nc + lax.axis_index("c")
        base = wid * b_per_w
        pltpu.sync_copy(idx_hbm.at[pl.ds(base, b_per_w)], idx_v)

        def start(chunk, buf):
            off = pl.multiple_of(chunk * _CHUNK, _CHUNK)
            pltpu.async_copy(
                table_hbm.at[idx_v.at[pl.ds(off, _CHUNK)]],
                rows_v.at[buf],
                sems[buf],
            )

        def wait(buf):
            pltpu.make_async_copy(
                table_hbm.at[idx_v.at[pl.ds(0, _CHUNK)]],
                rows_v.at[buf],
                sems[buf],
            ).wait()

        start(0, 0)

        def body(g, carry):
            for b in range(2):
                chunk = 2 * g + b

                @pl.when(chunk + 1 < n_chunks)
                def _():
                    start(chunk + 1, 1 - b)

                wait(b)
                out_off = pl.multiple_of(base + chunk * _CHUNK, _CHUNK)
                pltpu.sync_copy(rows_v.at[b], out_hbm.at[pl.ds(out_off, _CHUNK)])
            return carry

        lax.fori_loop(0, n_chunks // 2, body, 0)

    return gather_kernel


def kernel(positions, table):
    out_shape = positions.shape + (table.shape[1],)
    flat = jnp.clip(positions.reshape(-1), 0, _MAX_LEN - 1)
    out = _make_gather(flat.shape[0], table.shape[0], table.shape[1])(flat, table)
    return out.reshape(out_shape)
